# baseline (speedup 1.0000x reference)
import jax
import jax.numpy as jnp
from jax.experimental import pallas as pl
from jax.experimental.pallas import tpu as pltpu

N_FILTERS = 16
GRAPHLET = 3
N_GROUPS = 32
REL_DIM = 16
PROJ_DIM = 16
KEY_DIM = 16
BETA = KEY_DIM ** (-0.5)
NQ = N_GROUPS * GRAPHLET


ROWS_PER_STEP = 2


def _fused_kernel(x_ref, q_ref, pos_ref, wk_ref, wp_ref, m_ref, o_ref, pq_ref):
    i = pl.program_id(1)

    @pl.when(i == 0)
    def _init():
        pq_ref[...] = BETA * jax.lax.dot_general(
            q_ref[...], pos_ref[...], (((1,), (1,)), ((), ())),
            preferred_element_type=jnp.float32)

    qw = BETA * jax.lax.dot_general(
        q_ref[...], wk_ref[...], (((1,), (1,)), ((), ())),
        preferred_element_type=jnp.float32)
    rows = jax.lax.broadcasted_iota(jnp.int32, (N_GROUPS, NQ), 0)
    cols = jax.lax.broadcasted_iota(jnp.int32, (N_GROUPS, NQ), 1)
    sels = [(cols == GRAPHLET * rows + g).astype(jnp.float32)
            for g in range(GRAPHLET)]
    for r in range(ROWS_PER_STEP):
        x = x_ref[r]
        logits = jax.lax.dot_general(
            qw, x, (((1,), (1,)), ((), ())),
            preferred_element_type=jnp.float32) + pq_ref[...]
        mx = jnp.max(logits, axis=1, keepdims=True)
        e = jnp.exp(logits - mx)
        denom = jnp.sum(e, axis=1, keepdims=True)
        attn = jnp.dot(e, x, preferred_element_type=jnp.float32) / denom
        zs = []
        for g in range(GRAPHLET):
            attn_g = jnp.dot(sels[g], attn, preferred_element_type=jnp.float32)
            zs.append(jnp.dot(attn_g, wp_ref[...],
                              preferred_element_type=jnp.float32))
        acc = jnp.zeros((N_GROUPS, N_FILTERS), dtype=jnp.float32)
        for g in range(GRAPHLET):
            for h in range(GRAPHLET):
                w = zs[g] * zs[h]
                acc = acc + jnp.dot(w, m_ref[GRAPHLET * g + h],
                                    preferred_element_type=jnp.float32)
        o_ref[r] = acc


@jax.jit
def kernel(inputs, filters, group_queries, pos_emb, Wk, Wp):
    b, n, d = inputs.shape
    m = jnp.repeat(filters.transpose(1, 2, 3, 0), PROJ_DIM, axis=2)
    m = m.reshape(GRAPHLET * GRAPHLET, REL_DIM * PROJ_DIM, N_FILTERS)
    n_cores = 2
    inner = b // (ROWS_PER_STEP * n_cores)
    return pl.pallas_call(
        _fused_kernel,
        grid=(n_cores, inner),
        in_specs=[
            pl.BlockSpec((ROWS_PER_STEP, n, d),
                         lambda c, i: (c * inner + i, 0, 0)),
            pl.BlockSpec((NQ, KEY_DIM), lambda c, i: (0, 0)),
            pl.BlockSpec((n, KEY_DIM), lambda c, i: (0, 0)),
            pl.BlockSpec((d, KEY_DIM), lambda c, i: (0, 0)),
            pl.BlockSpec((d, REL_DIM * PROJ_DIM), lambda c, i: (0, 0)),
            pl.BlockSpec((GRAPHLET * GRAPHLET, REL_DIM * PROJ_DIM, N_FILTERS),
                         lambda c, i: (0, 0, 0)),
        ],
        out_specs=pl.BlockSpec((ROWS_PER_STEP, N_GROUPS, N_FILTERS),
                               lambda c, i: (c * inner + i, 0, 0)),
        out_shape=jax.ShapeDtypeStruct((b, N_GROUPS, N_FILTERS), jnp.float32),
        scratch_shapes=[pltpu.VMEM((NQ, n), jnp.float32)],
        compiler_params=pltpu.CompilerParams(
            dimension_semantics=("parallel", "arbitrary")),
    )(inputs, group_queries, pos_emb, Wk, Wp, m)


# speedup vs baseline: 1.0101x; 1.0101x over previous
import jax
import jax.numpy as jnp
from jax.experimental import pallas as pl
from jax.experimental.pallas import tpu as pltpu

N_FILTERS = 16
GRAPHLET = 3
N_GROUPS = 32
REL_DIM = 16
PROJ_DIM = 16
KEY_DIM = 16
BETA = KEY_DIM ** (-0.5)
NQ = N_GROUPS * GRAPHLET


ROWS_PER_STEP = 2


def _fused_kernel(x_ref, q_ref, pos_ref, wk_ref, wp_ref, m_ref, o_ref, pq_ref):
    i = pl.program_id(0)

    @pl.when(i == 0)
    def _init():
        pq_ref[...] = BETA * jax.lax.dot_general(
            q_ref[...], pos_ref[...], (((1,), (1,)), ((), ())),
            preferred_element_type=jnp.float32)

    qw = BETA * jax.lax.dot_general(
        q_ref[...], wk_ref[...], (((1,), (1,)), ((), ())),
        preferred_element_type=jnp.float32)
    rows = jax.lax.broadcasted_iota(jnp.int32, (N_GROUPS, NQ), 0)
    cols = jax.lax.broadcasted_iota(jnp.int32, (N_GROUPS, NQ), 1)
    sels = [(cols == GRAPHLET * rows + g).astype(jnp.float32)
            for g in range(GRAPHLET)]
    for r in range(ROWS_PER_STEP):
        x = x_ref[r]
        logits = jax.lax.dot_general(
            qw, x, (((1,), (1,)), ((), ())),
            preferred_element_type=jnp.float32) + pq_ref[...]
        mx = jnp.max(logits, axis=1, keepdims=True)
        e = jnp.exp(logits - mx)
        denom = jnp.sum(e, axis=1, keepdims=True)
        attn = jnp.dot(e, x, preferred_element_type=jnp.float32) / denom
        zs = []
        for g in range(GRAPHLET):
            attn_g = jnp.dot(sels[g], attn, preferred_element_type=jnp.float32)
            zs.append(jnp.dot(attn_g, wp_ref[...],
                              preferred_element_type=jnp.float32))
        acc = jnp.zeros((N_GROUPS, N_FILTERS), dtype=jnp.float32)
        for g in range(GRAPHLET):
            for h in range(GRAPHLET):
                w = zs[g] * zs[h]
                acc = acc + jnp.dot(w, m_ref[GRAPHLET * g + h],
                                    preferred_element_type=jnp.float32)
        o_ref[r] = acc


@jax.jit
def kernel(inputs, filters, group_queries, pos_emb, Wk, Wp):
    b, n, d = inputs.shape
    m = jnp.repeat(filters.transpose(1, 2, 3, 0), PROJ_DIM, axis=2)
    m = m.reshape(GRAPHLET * GRAPHLET, REL_DIM * PROJ_DIM, N_FILTERS)
    return pl.pallas_call(
        _fused_kernel,
        grid=(b // ROWS_PER_STEP,),
        in_specs=[
            pl.BlockSpec((ROWS_PER_STEP, n, d), lambda i: (i, 0, 0)),
            pl.BlockSpec((NQ, KEY_DIM), lambda i: (0, 0)),
            pl.BlockSpec((n, KEY_DIM), lambda i: (0, 0)),
            pl.BlockSpec((d, KEY_DIM), lambda i: (0, 0)),
            pl.BlockSpec((d, REL_DIM * PROJ_DIM), lambda i: (0, 0)),
            pl.BlockSpec((GRAPHLET * GRAPHLET, REL_DIM * PROJ_DIM, N_FILTERS),
                         lambda i: (0, 0, 0)),
        ],
        out_specs=pl.BlockSpec((ROWS_PER_STEP, N_GROUPS, N_FILTERS),
                               lambda i: (i, 0, 0)),
        out_shape=jax.ShapeDtypeStruct((b, N_GROUPS, N_FILTERS), jnp.float32),
        scratch_shapes=[pltpu.VMEM((NQ, n), jnp.float32)],
    )(inputs, group_queries, pos_emb, Wk, Wp, m)
